# baseline (speedup 1.0000x reference)
import jax
import jax.numpy as jnp
from jax import lax
from jax.experimental import pallas as pl
from jax.experimental.pallas import tpu as pltpu

_NUM_EMB = 1024
_DIM = 64
_TOP_K = 3
_COMMIT = 0.25


def _vq_body(x_ref, e_ref, dist_ref, idx_ref, q_ref, loss_ref, perp_ref,
             cnt_ref, sse_ref, *, nb, n_total, num_blocks):
    i = pl.program_id(0)
    x = x_ref[...]
    e = e_ref[...]
    x2 = jnp.sum(x * x, axis=1, keepdims=True)
    e2 = jnp.sum(e * e, axis=1, keepdims=True).T
    xe = jnp.dot(x, e.T, preferred_element_type=jnp.float32)
    d = x2 + e2 - 2.0 * xe
    dist_ref[...] = d

    iota = lax.broadcasted_iota(jnp.int32, (nb, _NUM_EMB), 1)
    work = d
    cnt = jnp.zeros((1, _NUM_EMB), jnp.float32)
    sse = jnp.zeros((), jnp.float32)
    idx_cols = []
    q_slabs = []
    for _ in range(_TOP_K):
        m = jnp.min(work, axis=1, keepdims=True)
        idxj = jnp.min(jnp.where(work == m, iota, _NUM_EMB),
                       axis=1, keepdims=True)
        hit = iota == idxj
        oh = hit.astype(jnp.float32)
        qj = jnp.dot(oh, e, preferred_element_type=jnp.float32)
        idx_cols.append(idxj)
        q_slabs.append(qj[:, None, :])
        cnt = cnt + jnp.sum(oh, axis=0, keepdims=True)
        diff = qj - x
        sse = sse + jnp.sum(diff * diff)
        work = jnp.where(hit, jnp.inf, work)

    idx_ref[...] = jnp.concatenate(idx_cols, axis=1)
    q_ref[...] = jnp.concatenate(q_slabs, axis=1)

    sse11 = sse.reshape(1, 1)

    @pl.when(i == 0)
    def _init():
        cnt_ref[...] = cnt
        sse_ref[...] = sse11
        loss_ref[...] = jnp.zeros((1, 1), jnp.float32)
        perp_ref[...] = jnp.zeros((1, 1), jnp.float32)

    @pl.when(i > 0)
    def _acc():
        cnt_ref[...] = cnt_ref[...] + cnt
        sse_ref[...] = sse_ref[...] + sse11

    @pl.when(i == num_blocks - 1)
    def _fin():
        avg = cnt_ref[...] / n_total
        ent = jnp.sum(avg * jnp.log(avg + 1e-10), keepdims=True)
        perp_ref[...] = jnp.exp(-ent).reshape(1, 1)
        loss_ref[...] = sse_ref[...] * (_COMMIT / (n_total * _TOP_K * _DIM))


def kernel(inputs, embedding_weight):
    B, T, C = inputs.shape
    N = B * T
    K = embedding_weight.shape[0]
    nb = 2304
    num_blocks = N // nb
    flat = inputs.reshape(N, C)

    import functools
    dist, idx, q, loss, perp = pl.pallas_call(
        functools.partial(_vq_body, nb=nb, n_total=float(N),
                          num_blocks=num_blocks),
        grid=(num_blocks,),
        in_specs=[
            pl.BlockSpec((nb, C), lambda i: (i, 0)),
            pl.BlockSpec((K, C), lambda i: (0, 0)),
        ],
        out_specs=[
            pl.BlockSpec((nb, K), lambda i: (i, 0)),
            pl.BlockSpec((nb, _TOP_K), lambda i: (i, 0)),
            pl.BlockSpec((nb, _TOP_K, C), lambda i: (i, 0, 0)),
            pl.BlockSpec((1, 1), lambda i: (0, 0)),
            pl.BlockSpec((1, 1), lambda i: (0, 0)),
        ],
        out_shape=[
            jax.ShapeDtypeStruct((N, K), jnp.float32),
            jax.ShapeDtypeStruct((N, _TOP_K), jnp.int32),
            jax.ShapeDtypeStruct((N, _TOP_K, C), jnp.float32),
            jax.ShapeDtypeStruct((1, 1), jnp.float32),
            jax.ShapeDtypeStruct((1, 1), jnp.float32),
        ],
        scratch_shapes=[
            pltpu.VMEM((1, K), jnp.float32),
            pltpu.VMEM((1, 1), jnp.float32),
        ],
    )(flat, embedding_weight)

    quantized_st = q.reshape(B, T, _TOP_K, C)
    return (loss[0, 0], quantized_st, perp[0, 0], idx, dist)


# speedup vs baseline: 1.1349x; 1.1349x over previous
import jax
import jax.numpy as jnp
from jax import lax
from jax.experimental import pallas as pl
from jax.experimental.pallas import tpu as pltpu

_NUM_EMB = 1024
_DIM = 64
_TOP_K = 3
_COMMIT = 0.25


def _vq_body(x_ref, e_ref, dist_ref, idx_ref, q_ref, loss_ref, perp_ref,
             cnt_ref, sse_ref, *, nb, n_total, num_blocks):
    i = pl.program_id(0)
    x = x_ref[...]
    e = e_ref[...]
    x2 = jnp.sum(x * x, axis=1, keepdims=True)
    e2 = jnp.sum(e * e, axis=1, keepdims=True).T
    xe = jnp.dot(-2.0 * x, e.T, preferred_element_type=jnp.float32)
    d = xe + (x2 + e2)
    dist_ref[...] = d

    iota = lax.broadcasted_iota(
        jnp.int32, (nb, _NUM_EMB), 1).astype(jnp.float32)
    work = d
    sse = jnp.zeros((), jnp.float32)
    idx_cols = []
    q_slabs = []
    for _ in range(_TOP_K):
        m = jnp.min(work, axis=1, keepdims=True)
        idxj = jnp.min(jnp.where(work == m, iota, float(_NUM_EMB)),
                       axis=1, keepdims=True)
        hit = iota == idxj
        oh = hit.astype(jnp.float32)
        qj = jnp.dot(oh, e, preferred_element_type=jnp.float32)
        idx_cols.append(idxj.astype(jnp.int32))
        q_slabs.append(qj[:, None, :])
        diff = qj - x
        sse = sse + jnp.sum(diff * diff)
        work = jnp.where(hit, jnp.inf, work)

    cnt = jnp.sum((work == jnp.inf).astype(jnp.float32), axis=0,
                  keepdims=True)

    idx_ref[...] = jnp.concatenate(idx_cols, axis=1)
    q_ref[...] = jnp.concatenate(q_slabs, axis=1)

    sse11 = sse.reshape(1, 1)

    @pl.when(i == 0)
    def _init():
        cnt_ref[...] = cnt
        sse_ref[...] = sse11
        loss_ref[...] = jnp.zeros((1, 1), jnp.float32)
        perp_ref[...] = jnp.zeros((1, 1), jnp.float32)

    @pl.when(i > 0)
    def _acc():
        cnt_ref[...] = cnt_ref[...] + cnt
        sse_ref[...] = sse_ref[...] + sse11

    @pl.when(i == num_blocks - 1)
    def _fin():
        avg = cnt_ref[...] / n_total
        ent = jnp.sum(avg * jnp.log(avg + 1e-10), keepdims=True)
        perp_ref[...] = jnp.exp(-ent).reshape(1, 1)
        loss_ref[...] = sse_ref[...] * (_COMMIT / (n_total * _TOP_K * _DIM))


def kernel(inputs, embedding_weight):
    B, T, C = inputs.shape
    N = B * T
    K = embedding_weight.shape[0]
    nb = 1024
    num_blocks = N // nb
    flat = inputs.reshape(N, C)

    import functools
    dist, idx, q, loss, perp = pl.pallas_call(
        functools.partial(_vq_body, nb=nb, n_total=float(N),
                          num_blocks=num_blocks),
        grid=(num_blocks,),
        in_specs=[
            pl.BlockSpec((nb, C), lambda i: (i, 0)),
            pl.BlockSpec((K, C), lambda i: (0, 0)),
        ],
        out_specs=[
            pl.BlockSpec((nb, K), lambda i: (i, 0)),
            pl.BlockSpec((nb, _TOP_K), lambda i: (i, 0)),
            pl.BlockSpec((nb, _TOP_K, C), lambda i: (i, 0, 0)),
            pl.BlockSpec((1, 1), lambda i: (0, 0)),
            pl.BlockSpec((1, 1), lambda i: (0, 0)),
        ],
        out_shape=[
            jax.ShapeDtypeStruct((N, K), jnp.float32),
            jax.ShapeDtypeStruct((N, _TOP_K), jnp.int32),
            jax.ShapeDtypeStruct((N, _TOP_K, C), jnp.float32),
            jax.ShapeDtypeStruct((1, 1), jnp.float32),
            jax.ShapeDtypeStruct((1, 1), jnp.float32),
        ],
        scratch_shapes=[
            pltpu.VMEM((1, K), jnp.float32),
            pltpu.VMEM((1, 1), jnp.float32),
        ],
    )(flat, embedding_weight)

    quantized_st = q.reshape(B, T, _TOP_K, C)
    return (loss[0, 0], quantized_st, perp[0, 0], idx, dist)


# speedup vs baseline: 1.2083x; 1.0646x over previous
import jax
import jax.numpy as jnp
from jax import lax
from jax.experimental import pallas as pl
from jax.experimental.pallas import tpu as pltpu

_NUM_EMB = 1024
_DIM = 64
_TOP_K = 3
_COMMIT = 0.25


def _vq_body(x_ref, e_ref, dist_ref, idx_ref, q_ref, loss_ref, perp_ref,
             cnt_ref, sse_ref, *, nb, n_total, num_blocks):
    i = pl.program_id(0)
    x = x_ref[...]
    e = e_ref[...]
    x2 = jnp.sum(x * x, axis=1, keepdims=True)
    e2 = jnp.sum(e * e, axis=1, keepdims=True).T
    xe = jnp.dot(-2.0 * x, e.T, preferred_element_type=jnp.float32)
    d = xe + (x2 + e2)
    dist_ref[...] = d

    iota = lax.broadcasted_iota(
        jnp.int32, (nb, _NUM_EMB), 1).astype(jnp.float32)
    work = d
    sse = jnp.zeros((), jnp.float32)
    idx_cols = []
    q_slabs = []
    for _ in range(_TOP_K):
        m = jnp.min(work, axis=1, keepdims=True)
        idxj = jnp.min(jnp.where(work == m, iota, float(_NUM_EMB)),
                       axis=1, keepdims=True)
        hit = iota == idxj
        oh = hit.astype(jnp.float32)
        qj = jnp.dot(oh, e, preferred_element_type=jnp.float32)
        idx_cols.append(idxj.astype(jnp.int32))
        q_slabs.append(qj[:, None, :])
        diff = qj - x
        sse = sse + jnp.sum(diff * diff)
        work = jnp.where(hit, jnp.inf, work)

    cnt = jnp.sum((work == jnp.inf).astype(jnp.float32), axis=0,
                  keepdims=True)

    idx_ref[...] = jnp.concatenate(idx_cols, axis=1)
    q_ref[...] = jnp.concatenate(q_slabs, axis=1)[None]

    sse11 = sse.reshape(1, 1)

    @pl.when(i == 0)
    def _init():
        cnt_ref[...] = cnt
        sse_ref[...] = sse11
        loss_ref[...] = jnp.zeros((1, 1), jnp.float32)
        perp_ref[...] = jnp.zeros((1, 1), jnp.float32)

    @pl.when(i > 0)
    def _acc():
        cnt_ref[...] = cnt_ref[...] + cnt
        sse_ref[...] = sse_ref[...] + sse11

    @pl.when(i == num_blocks - 1)
    def _fin():
        avg = cnt_ref[...] / n_total
        ent = jnp.sum(avg * jnp.log(avg + 1e-10), keepdims=True)
        perp_ref[...] = jnp.exp(-ent).reshape(1, 1)
        loss_ref[...] = sse_ref[...] * (_COMMIT / (n_total * _TOP_K * _DIM))


def kernel(inputs, embedding_weight):
    B, T, C = inputs.shape
    N = B * T
    K = embedding_weight.shape[0]
    nb = T
    num_blocks = N // nb
    flat = inputs.reshape(N, C)

    import functools
    dist, idx, q, loss, perp = pl.pallas_call(
        functools.partial(_vq_body, nb=nb, n_total=float(N),
                          num_blocks=num_blocks),
        grid=(num_blocks,),
        in_specs=[
            pl.BlockSpec((nb, C), lambda i: (i, 0)),
            pl.BlockSpec((K, C), lambda i: (0, 0)),
        ],
        out_specs=[
            pl.BlockSpec((nb, K), lambda i: (i, 0)),
            pl.BlockSpec((nb, _TOP_K), lambda i: (i, 0)),
            pl.BlockSpec((1, nb, _TOP_K, C), lambda i: (i, 0, 0, 0)),
            pl.BlockSpec((1, 1), lambda i: (0, 0)),
            pl.BlockSpec((1, 1), lambda i: (0, 0)),
        ],
        out_shape=[
            jax.ShapeDtypeStruct((N, K), jnp.float32),
            jax.ShapeDtypeStruct((N, _TOP_K), jnp.int32),
            jax.ShapeDtypeStruct((B, T, _TOP_K, C), jnp.float32),
            jax.ShapeDtypeStruct((1, 1), jnp.float32),
            jax.ShapeDtypeStruct((1, 1), jnp.float32),
        ],
        scratch_shapes=[
            pltpu.VMEM((1, K), jnp.float32),
            pltpu.VMEM((1, 1), jnp.float32),
        ],
    )(flat, embedding_weight)

    return (loss[0, 0], q, perp[0, 0], idx, dist)


# speedup vs baseline: 1.6558x; 1.3703x over previous
import jax
import jax.numpy as jnp
from jax import lax
from jax.experimental import pallas as pl
from jax.experimental.pallas import tpu as pltpu

_NUM_EMB = 1024
_DIM = 64
_TOP_K = 3
_COMMIT = 0.25


def _vq_body(x_ref, e_ref, dist_ref, idx_ref, q_ref, loss_ref, perp_ref,
             cnt_ref, sse_ref, *, nb, n_total, num_blocks):
    i = pl.program_id(0)
    x = x_ref[...]
    e = e_ref[...]
    x2 = jnp.sum(x * x, axis=1, keepdims=True)
    e2 = jnp.sum(e * e, axis=1, keepdims=True).T
    xe = jnp.dot(-2.0 * x, e.T, preferred_element_type=jnp.float32)
    d = xe + (x2 + e2)
    dist_ref[...] = d

    iota = lax.broadcasted_iota(
        jnp.int32, (nb, _NUM_EMB), 1).astype(jnp.float32)
    work = d
    sse = jnp.zeros((), jnp.float32)
    idx_cols = []
    q_slabs = []
    for _ in range(_TOP_K):
        m = jnp.min(work, axis=1, keepdims=True)
        idxj = jnp.min(jnp.where(work == m, iota, float(_NUM_EMB)),
                       axis=1, keepdims=True)
        hit = iota == idxj
        oh = hit.astype(jnp.float32)
        qj = jnp.dot(oh, e, preferred_element_type=jnp.float32)
        idx_cols.append(idxj.astype(jnp.int32))
        q_slabs.append(qj.T[None])
        diff = qj - x
        sse = sse + jnp.sum(diff * diff)
        work = jnp.where(hit, jnp.inf, work)

    cnt = jnp.sum((work == jnp.inf).astype(jnp.float32), axis=0,
                  keepdims=True)

    idx_ref[...] = jnp.concatenate(idx_cols, axis=1)
    q_ref[...] = jnp.concatenate(q_slabs, axis=0)[None]

    sse11 = sse.reshape(1, 1)

    @pl.when(i == 0)
    def _init():
        cnt_ref[...] = cnt
        sse_ref[...] = sse11
        loss_ref[...] = jnp.zeros((1, 1), jnp.float32)
        perp_ref[...] = jnp.zeros((1, 1), jnp.float32)

    @pl.when(i > 0)
    def _acc():
        cnt_ref[...] = cnt_ref[...] + cnt
        sse_ref[...] = sse_ref[...] + sse11

    @pl.when(i == num_blocks - 1)
    def _fin():
        avg = cnt_ref[...] / n_total
        ent = jnp.sum(avg * jnp.log(avg + 1e-10), keepdims=True)
        perp_ref[...] = jnp.exp(-ent).reshape(1, 1)
        loss_ref[...] = sse_ref[...] * (_COMMIT / (n_total * _TOP_K * _DIM))


def kernel(inputs, embedding_weight):
    B, T, C = inputs.shape
    N = B * T
    K = embedding_weight.shape[0]
    nb = T
    num_blocks = N // nb
    flat = inputs.reshape(N, C)

    import functools
    dist, idx, q, loss, perp = pl.pallas_call(
        functools.partial(_vq_body, nb=nb, n_total=float(N),
                          num_blocks=num_blocks),
        grid=(num_blocks,),
        in_specs=[
            pl.BlockSpec((nb, C), lambda i: (i, 0)),
            pl.BlockSpec((K, C), lambda i: (0, 0)),
        ],
        out_specs=[
            pl.BlockSpec((nb, K), lambda i: (i, 0)),
            pl.BlockSpec((nb, _TOP_K), lambda i: (i, 0)),
            pl.BlockSpec((1, _TOP_K, C, nb), lambda i: (i, 0, 0, 0)),
            pl.BlockSpec((1, 1), lambda i: (0, 0)),
            pl.BlockSpec((1, 1), lambda i: (0, 0)),
        ],
        out_shape=[
            jax.ShapeDtypeStruct((N, K), jnp.float32),
            jax.ShapeDtypeStruct((N, _TOP_K), jnp.int32),
            jax.ShapeDtypeStruct((B, _TOP_K, C, T), jnp.float32),
            jax.ShapeDtypeStruct((1, 1), jnp.float32),
            jax.ShapeDtypeStruct((1, 1), jnp.float32),
        ],
        scratch_shapes=[
            pltpu.VMEM((1, K), jnp.float32),
            pltpu.VMEM((1, 1), jnp.float32),
        ],
    )(flat, embedding_weight)

    quantized_st = jnp.transpose(q, (0, 3, 1, 2))
    return (loss[0, 0], quantized_st, perp[0, 0], idx, dist)


# speedup vs baseline: 2.2553x; 1.3621x over previous
import functools

import jax
import jax.numpy as jnp
from jax import lax
from jax.experimental import pallas as pl
from jax.experimental.pallas import tpu as pltpu

_NUM_EMB = 1024
_DIM = 64
_TOP_K = 3
_COMMIT = 0.25


def _vq_body(xt_ref, et_ref, dist_ref, idx_ref, q_ref, loss_ref, perp_ref,
             cnt_ref, sse_ref, *, nb, n_total, num_blocks):
    i = pl.program_id(0)
    xt = xt_ref[0]
    et = et_ref[...]

    ones_n = jnp.ones((1, nb), jnp.float32)
    ones_k = jnp.ones((1, _NUM_EMB), jnp.float32)
    x2 = jnp.sum(xt * xt, axis=0, keepdims=True)
    e2 = jnp.sum(et * et, axis=0, keepdims=True)
    lhs = jnp.concatenate([-2.0 * xt, x2, ones_n], axis=0)
    rhs = jnp.concatenate([et, ones_k, e2], axis=0)
    d = lax.dot_general(lhs, rhs, (((0,), (0,)), ((), ())),
                        preferred_element_type=jnp.float32)
    dist_ref[...] = d

    iota = lax.broadcasted_iota(
        jnp.int32, (nb, _NUM_EMB), 1).astype(jnp.float32)
    work = d
    sse = jnp.zeros((), jnp.float32)
    idx_rows = []
    q_slabs = []
    for _ in range(_TOP_K):
        m = jnp.min(work, axis=1, keepdims=True)
        idxj = jnp.min(jnp.where(work == m, iota, float(_NUM_EMB)),
                       axis=1, keepdims=True)
        hit = iota == idxj
        oh = hit.astype(jnp.float32)
        qjt = lax.dot_general(et, oh, (((1,), (1,)), ((), ())),
                              preferred_element_type=jnp.float32)
        idx_rows.append(idxj.T)
        q_slabs.append(qjt[None])
        diff = qjt - xt
        sse = sse + jnp.sum(diff * diff)
        work = jnp.where(hit, jnp.inf, work)

    cnt = jnp.sum((work == jnp.inf).astype(jnp.float32), axis=0,
                  keepdims=True)

    idx_ref[...] = jnp.concatenate(idx_rows, axis=0)[None].astype(jnp.int32)
    q_ref[...] = jnp.concatenate(q_slabs, axis=0)[None]

    sse11 = sse.reshape(1, 1)

    @pl.when(i == 0)
    def _init():
        cnt_ref[...] = cnt
        sse_ref[...] = sse11
        loss_ref[...] = jnp.zeros((1, 1), jnp.float32)
        perp_ref[...] = jnp.zeros((1, 1), jnp.float32)

    @pl.when(i > 0)
    def _acc():
        cnt_ref[...] = cnt_ref[...] + cnt
        sse_ref[...] = sse_ref[...] + sse11

    @pl.when(i == num_blocks - 1)
    def _fin():
        avg = cnt_ref[...] / n_total
        ent = jnp.sum(avg * jnp.log(avg + 1e-10), keepdims=True)
        perp_ref[...] = jnp.exp(-ent).reshape(1, 1)
        loss_ref[...] = sse_ref[...] * (_COMMIT / (n_total * _TOP_K * _DIM))


def kernel(inputs, embedding_weight):
    B, T, C = inputs.shape
    N = B * T
    K = embedding_weight.shape[0]
    nb = T
    num_blocks = B

    xt = jnp.transpose(inputs, (0, 2, 1))
    et = embedding_weight.T

    dist, idx3, q, loss, perp = pl.pallas_call(
        functools.partial(_vq_body, nb=nb, n_total=float(N),
                          num_blocks=num_blocks),
        grid=(num_blocks,),
        in_specs=[
            pl.BlockSpec((1, C, nb), lambda i: (i, 0, 0)),
            pl.BlockSpec((C, K), lambda i: (0, 0)),
        ],
        out_specs=[
            pl.BlockSpec((nb, K), lambda i: (i, 0)),
            pl.BlockSpec((1, _TOP_K, nb), lambda i: (i, 0, 0)),
            pl.BlockSpec((1, _TOP_K, C, nb), lambda i: (i, 0, 0, 0)),
            pl.BlockSpec((1, 1), lambda i: (0, 0)),
            pl.BlockSpec((1, 1), lambda i: (0, 0)),
        ],
        out_shape=[
            jax.ShapeDtypeStruct((N, K), jnp.float32),
            jax.ShapeDtypeStruct((B, _TOP_K, T), jnp.int32),
            jax.ShapeDtypeStruct((B, _TOP_K, C, T), jnp.float32),
            jax.ShapeDtypeStruct((1, 1), jnp.float32),
            jax.ShapeDtypeStruct((1, 1), jnp.float32),
        ],
        scratch_shapes=[
            pltpu.VMEM((1, K), jnp.float32),
            pltpu.VMEM((1, 1), jnp.float32),
        ],
    )(xt, et)

    quantized_st = jnp.transpose(q, (0, 3, 1, 2))
    idx = jnp.transpose(idx3, (0, 2, 1)).reshape(N, _TOP_K)
    return (loss[0, 0], quantized_st, perp[0, 0], idx, dist)
